# baseline (speedup 1.0000x reference)
import functools

import jax
import jax.numpy as jnp
from jax import lax
from jax.experimental import pallas as pl
from jax.experimental.pallas import tpu as pltpu
from jax.experimental.pallas import tpu_sc as plsc

VOCAB = 1000000
EMBED_DIM = 32

NC = 2
NS = 16
NW = NC * NS

B = 4096
T = 200
RT = T // 8
CT = B // 128
N_PAIRS = (RT * CT) // NW
NBUF = 4


def _make_kernel():
  mesh = plsc.VectorSubcoreMesh(
      core_axis_name="c", subcore_axis_name="s", num_cores=NC,
      num_subcores=NS)

  @functools.partial(
      pl.kernel,
      out_type=jax.ShapeDtypeStruct((T, 4, CT, 8, 128), jnp.float32),
      mesh=mesh,
      scratch_types=[
          pltpu.VMEM((2, 8, 128), jnp.int32),
          pltpu.VMEM((NBUF, 128, EMBED_DIM), jnp.float32),
          pltpu.VMEM((NBUF, 4, 8, 128), jnp.float32),
          pltpu.SemaphoreType.DMA((2,)),
          pltpu.SemaphoreType.DMA((NBUF,)),
          pltpu.SemaphoreType.DMA((NBUF,)),
      ],
      compiler_params=pltpu.CompilerParams(
          use_tc_tiling_on_sc=False, needs_layout_passes=False,
          disable_bounds_checks=True),
  )
  def gather_kernel(idx_hbm, table_hbm, out_hbm, xbuf, rows_v, z_v,
                    sem_i, sem_g, sem_o):
    wid = lax.axis_index("s") * NC + lax.axis_index("c")
    p0 = wid * N_PAIRS
    iot = lax.iota(jnp.int32, 16)

    N = N_PAIRS * 8

    def idx_load(q):
      p = p0 + q
      return pltpu.make_async_copy(
          idx_hbm.at[p // CT, p % CT], xbuf.at[(p0 + q) % 2],
          sem_i.at[(p0 + q) % 2])

    def gather(n):
      q, rs, b = n // 8, n % 8, n % NBUF
      return pltpu.make_async_copy(
          table_hbm.at[xbuf.at[(p0 + q) % 2, rs]], rows_v.at[b],
          sem_g.at[b])

    def writeback(n):
      q, rs, b = n // 8, n % 8, n % NBUF
      p = p0 + q
      t = (p // CT) * 8 + rs
      return pltpu.make_async_copy(
          z_v.at[b], out_hbm.at[t, :, p % CT], sem_o.at[b])

    def transpose(b):
      rows = rows_v.at[b]
      for kt in range(4):
        for ks in range(8):
          col = jnp.full((16,), kt * 8 + ks, jnp.int32)
          for blg in range(8):
            row = iot + blg * 16
            vals = plsc.load_gather(rows, [row, col])
            z_v[b, kt, ks, pl.ds(blg * 16, 16)] = vals

    idx_load(0).start()
    idx_load(0).wait()
    for n in range(3):
      gather(n).start()

    @pl.loop(0, N)
    def _chunk(n):
      @pl.when(jnp.logical_and(n % 8 == 0, n < N - 8))
      def _():
        idx_load(n // 8 + 1).start()

      @pl.when(jnp.logical_and(n % 8 == 5, n < N - 8))
      def _():
        idx_load(n // 8 + 1).wait()

      @pl.when(n >= 1)
      def _():
        writeback(n - 1).wait()

      @pl.when(n + 3 < N)
      def _():
        gather(n + 3).start()

      gather(n).wait()
      transpose(n % NBUF)
      writeback(n).start()

    writeback(N - 1).wait()

  return gather_kernel


_gather = _make_kernel()


@jax.jit
def kernel(token_indices, embedding_table):
  idx4 = (token_indices.T.astype(jnp.int32)
          .reshape(RT, 8, CT, 128).transpose(0, 2, 1, 3))
  z = _gather(idx4, embedding_table)
  return z.transpose(2, 4, 0, 1, 3).reshape(B, T, EMBED_DIM)


# speedup vs baseline: 1.1787x; 1.1787x over previous
import functools

import jax
import jax.numpy as jnp
from jax import lax
from jax.experimental import pallas as pl
from jax.experimental.pallas import tpu as pltpu
from jax.experimental.pallas import tpu_sc as plsc

VOCAB = 1000000
EMBED_DIM = 32

NC = 2
NS = 16
NW = NC * NS

B = 4096
T = 200
CHUNK = 512
BLK = B // CHUNK
N_CHUNKS = (T * BLK) // NW
PER_W = N_CHUNKS * CHUNK
IDXROWS = PER_W // B + 1
NBUF = 5


def _make_kernel():
  mesh = plsc.VectorSubcoreMesh(
      core_axis_name="c", subcore_axis_name="s", num_cores=NC,
      num_subcores=NS)

  @functools.partial(
      pl.kernel,
      out_type=jax.ShapeDtypeStruct((B, T, EMBED_DIM), jnp.float32),
      mesh=mesh,
      scratch_types=[
          pltpu.VMEM((IDXROWS, B), jnp.int32),
          pltpu.VMEM((NBUF, CHUNK, EMBED_DIM), jnp.float32),
          pltpu.SemaphoreType.DMA((NBUF,)),
          pltpu.SemaphoreType.DMA((NBUF,)),
      ],
      compiler_params=pltpu.CompilerParams(use_tc_tiling_on_sc=False),
  )
  def gather_kernel(idx_hbm, table_hbm, out_hbm, idx_all, rows_v, sem_g,
                    sem_o):
    wid = lax.axis_index("s") * NC + lax.axis_index("c")
    t0 = (wid * PER_W) // B
    off0 = wid * PER_W - t0 * B
    pltpu.sync_copy(idx_hbm.at[pl.ds(t0, IDXROWS)], idx_all)

    def gather(c, b):
      p = off0 + c * CHUNK
      return pltpu.make_async_copy(
          table_hbm.at[idx_all.at[p // B, pl.ds(p % B, CHUNK)]],
          rows_v.at[b], sem_g.at[b])

    def writeback(c, b):
      g = wid * N_CHUNKS + c
      return pltpu.make_async_copy(
          rows_v.at[b],
          out_hbm.at[pl.ds((g % BLK) * CHUNK, CHUNK), g // BLK],
          sem_o.at[b])

    for c in range(NBUF):
      gather(c, c).start()
    gather(0, 0).wait()
    writeback(0, 0).start()

    @pl.loop(1, N_CHUNKS - NBUF + 1, step=NBUF)
    def _grp(g):
      for i in range(NBUF):
        c = g + i
        b = (1 + i) % NBUF
        bp = (b - 1) % NBUF
        writeback(c - 1, bp).wait()
        gather(c + NBUF - 1, bp).start()
        gather(c, b).wait()
        writeback(c, b).start()

    for c in range(N_CHUNKS - NBUF + 1, N_CHUNKS):
      b = c % NBUF
      writeback(c - 1, (b - 1) % NBUF).wait()
      gather(c, b).wait()
      writeback(c, b).start()
    writeback(N_CHUNKS - 1, (N_CHUNKS - 1) % NBUF).wait()

  return gather_kernel


_gather = _make_kernel()


@jax.jit
def kernel(token_indices, embedding_table):
  idx_t = token_indices.T.astype(jnp.int32)
  return _gather(idx_t, embedding_table)
